# Initial kernel scaffold; baseline (speedup 1.0000x reference)
import jax
import jax.numpy as jnp
from jax.experimental import pallas as pl


def kernel(emb, W, root, bias, W1, b1, W2, b2, node_type, edge_index, edge_type, batch):
    raise NotImplementedError("write your pallas kernel here")


# speedup vs baseline: 12.2697x; 12.2697x over previous
import jax
import jax.numpy as jnp
from jax import lax
from jax.experimental import pallas as pl
from jax.experimental.pallas import tpu as pltpu
from jax.experimental.pallas import tpu_sc as plsc

N = 50000
E = 800000
H = 64
R = 4
T = 8
G = 64
C = 2

NC = 2
NS = 16
SUB = 128
RPC = 4
K = RPC * SUB
EP = 50 * NC * NS * K
PADE = EP - E
NROW_HBM = EP // SUB
TBL = 200704
ZSL = TBL // 8
NROWS = N + 32
ZT = 10
ZROWS = N // ZT

BN = 1000
NB = N // BN

f32 = jnp.float32
i32 = jnp.int32

_mesh = plsc.VectorSubcoreMesh(core_axis_name="c", subcore_axis_name="s")


def _sc_counts_body(didx_hbm, cnt_out0, cnt_out1, cnt_sh, dix_v, zbuf, ones_v):
    c = lax.axis_index("c")
    s = lax.axis_index("s")
    wid = s * NC + c
    for i in range(250):
        zbuf[pl.ds(i * 16, 16)] = jnp.zeros((16,), f32)
    for i in range(8):
        ones_v[pl.ds(i * 16, 16)] = jnp.ones((16,), f32)

    @pl.when(s < 8)
    def _zero():
        zbase = s * ZSL
        for i in range(6):
            pltpu.sync_copy(zbuf, cnt_sh.at[pl.ds(zbase + i * 4000, 4000)])
        pltpu.sync_copy(zbuf.at[pl.ds(0, ZSL - 24000)],
                        cnt_sh.at[pl.ds(zbase + 24000, ZSL - 24000)])

    plsc.subcore_barrier()

    rpw = NROW_HBM // (NC * NS)

    def chunk(i, carry):
        row = wid * rpw + i * RPC
        pltpu.sync_copy(didx_hbm.at[pl.ds(row, RPC)], dix_v)
        for q in range(RPC):
            pltpu.sync_copy(ones_v, cnt_sh.at[dix_v.at[q]], add=True)
        return carry

    lax.fori_loop(0, rpw // RPC, chunk, 0)
    plsc.subcore_barrier()

    @pl.when((s < 8) & (c == 0))
    def _wb0():
        pltpu.sync_copy(cnt_sh.at[pl.ds(s * ZSL, ZSL)],
                        cnt_out0.at[pl.ds(s * ZSL, ZSL)])

    @pl.when((s < 8) & (c == 1))
    def _wb1():
        pltpu.sync_copy(cnt_sh.at[pl.ds(s * ZSL, ZSL)],
                        cnt_out1.at[pl.ds(s * ZSL, ZSL)])


_sc_counts = pl.kernel(
    _sc_counts_body,
    out_type=(jax.ShapeDtypeStruct((TBL,), f32),
              jax.ShapeDtypeStruct((TBL,), f32)),
    mesh=_mesh,
    scratch_types=[
        pltpu.VMEM_SHARED((TBL,), f32),
        pltpu.VMEM((RPC, SUB), i32),
        pltpu.VMEM((4000,), f32),
        pltpu.VMEM((SUB,), f32),
    ],
    compiler_params=pltpu.CompilerParams(use_tc_tiling_on_sc=False),
)


def _sc_weights_body(didx_hbm, inv_hbm, w_out, dix_v, w_v, sem):
    c = lax.axis_index("c")
    s = lax.axis_index("s")
    wid = s * NC + c
    rpw = NROW_HBM // (NC * NS)

    def chunk(i, carry):
        row = wid * rpw + i * RPC
        pltpu.sync_copy(didx_hbm.at[pl.ds(row, RPC)], dix_v)
        for q in range(RPC):
            pltpu.async_copy(inv_hbm.at[dix_v.at[q]],
                             w_v.at[pl.ds(q * SUB, SUB)], sem).wait()
        base = pl.multiple_of(row * SUB, K)
        pltpu.sync_copy(w_v, w_out.at[pl.ds(base, K)])
        return carry

    lax.fori_loop(0, rpw // RPC, chunk, 0)


_sc_weights = pl.kernel(
    _sc_weights_body,
    out_type=jax.ShapeDtypeStruct((EP,), f32),
    mesh=_mesh,
    scratch_types=[
        pltpu.VMEM((RPC, SUB), i32),
        pltpu.VMEM((K,), f32),
        pltpu.SemaphoreType.DMA,
    ],
    compiler_params=pltpu.CompilerParams(use_tc_tiling_on_sc=False),
)


def _sc_agg_body(y0_hbm, y1_hbm, gidx_hbm, dst_hbm, w_hbm, agg_out,
                 agg_sh, gix_v, dix_v, w_v, rows_v, zb2, sem):
    c = lax.axis_index("c")
    s = lax.axis_index("s")

    for i in range(125):
        zb2[i, pl.ds(0, 16)] = jnp.zeros((16,), f32)
        zb2[i, pl.ds(16, 16)] = jnp.zeros((16,), f32)

    @pl.when(s < ZT)
    def _zero():
        for i in range(ZROWS // 125):
            pltpu.sync_copy(zb2, agg_sh.at[pl.ds(s * ZROWS + i * 125, 125)])

    plsc.subcore_barrier()

    rpt = NROW_HBM // NS

    def chunk(i, carry):
        row = s * rpt + i * RPC
        pltpu.sync_copy(gidx_hbm.at[pl.ds(row, RPC)], gix_v)
        pltpu.sync_copy(dst_hbm.at[pl.ds(row, RPC)], dix_v)
        base = pl.multiple_of(row * SUB, K)
        pltpu.sync_copy(w_hbm.at[pl.ds(base, K)], w_v)
        for q in range(RPC):
            @pl.when(c == 0)
            def _g0():
                pltpu.async_copy(y0_hbm.at[gix_v.at[q]],
                                 rows_v.at[pl.ds(q * SUB, SUB)], sem).wait()

            @pl.when(c == 1)
            def _g1():
                pltpu.async_copy(y1_hbm.at[gix_v.at[q]],
                                 rows_v.at[pl.ds(q * SUB, SUB)], sem).wait()

        dn = lax.GatherDimensionNumbers(offset_dims=(), collapsed_slice_dims=(0,),
                                        start_index_map=(0,))

        def wgrp(j, carry2):
            wv = w_v[pl.ds(j * 16, 16)]
            for t in range(16):
                e = j * 16 + t
                wb = lax.gather(wv, jnp.full((16, 1), t, i32), dn, (1,),
                                mode=lax.GatherScatterMode.PROMISE_IN_BOUNDS)
                r0 = rows_v[e, pl.ds(0, 16)]
                rows_v[e, pl.ds(0, 16)] = r0 * wb
                r1 = rows_v[e, pl.ds(16, 16)]
                rows_v[e, pl.ds(16, 16)] = r1 * wb
            return carry2

        lax.fori_loop(0, K // 16, wgrp, 0)
        for q in range(RPC):
            pltpu.sync_copy(rows_v.at[pl.ds(q * SUB, SUB)],
                            agg_sh.at[dix_v.at[q]], add=True)
        return carry

    lax.fori_loop(0, rpt // RPC, chunk, 0)
    plsc.subcore_barrier()

    @pl.when(s < ZT)
    def _wb():
        pltpu.sync_copy(agg_sh.at[pl.ds(s * ZROWS, ZROWS)],
                        agg_out.at[c, pl.ds(s * ZROWS, ZROWS)])


_sc_agg = pl.kernel(
    _sc_agg_body,
    out_type=jax.ShapeDtypeStruct((NC, N, 32), f32),
    mesh=_mesh,
    scratch_types=[
        pltpu.VMEM_SHARED((NROWS, 32), f32),
        pltpu.VMEM((RPC, SUB), i32),
        pltpu.VMEM((RPC, SUB), i32),
        pltpu.VMEM((K,), f32),
        pltpu.VMEM((K, 32), f32),
        pltpu.VMEM((125, 32), f32),
        pltpu.SemaphoreType.DMA,
    ],
    compiler_params=pltpu.CompilerParams(use_tc_tiling_on_sc=False),
)


def _tc_inv_body(c0_ref, c1_ref, inv_ref):
    inv_ref[...] = 1.0 / jnp.maximum(c0_ref[...] + c1_ref[...], 1.0)


_tc_inv = pl.pallas_call(
    _tc_inv_body,
    grid=(1,),
    in_specs=[
        pl.BlockSpec((TBL // 128, 128), lambda i: (0, 0)),
        pl.BlockSpec((TBL // 128, 128), lambda i: (0, 0)),
    ],
    out_specs=pl.BlockSpec((TBL // 128, 128), lambda i: (0, 0)),
    out_shape=jax.ShapeDtypeStruct((TBL // 128, 128), f32),
)


def _tc_layer1_body(nt_ref, emb_ref, a0_ref, a1_ref, root_ref, bias_ref,
                    y0_ref, y1_ref, out0_ref):
    nt = nt_ref[...]
    oh = (nt == lax.broadcasted_iota(i32, (1, T), 1)).astype(f32)
    x = jnp.dot(oh, emb_ref[...], preferred_element_type=f32)
    y0_ref[...] = jnp.dot(x, a0_ref[...], preferred_element_type=f32)
    y1_ref[...] = jnp.dot(x, a1_ref[...], preferred_element_type=f32)
    out0_ref[...] = jnp.dot(x, root_ref[...], preferred_element_type=f32) + bias_ref[...]


_tc_layer1 = pl.pallas_call(
    _tc_layer1_body,
    grid=(NB,),
    in_specs=[
        pl.BlockSpec((BN, 1), lambda i: (i, 0)),
        pl.BlockSpec((T, H), lambda i: (0, 0)),
        pl.BlockSpec((H, R * 32), lambda i: (0, 0)),
        pl.BlockSpec((H, R * 32), lambda i: (0, 0)),
        pl.BlockSpec((H, H), lambda i: (0, 0)),
        pl.BlockSpec((1, H), lambda i: (0, 0)),
    ],
    out_specs=[
        pl.BlockSpec((BN, R * 32), lambda i: (i, 0)),
        pl.BlockSpec((BN, R * 32), lambda i: (i, 0)),
        pl.BlockSpec((BN, H), lambda i: (i, 0)),
    ],
    out_shape=[
        jax.ShapeDtypeStruct((N, R * 32), f32),
        jax.ShapeDtypeStruct((N, R * 32), f32),
        jax.ShapeDtypeStruct((N, H), f32),
    ],
)


def _tc_layer2_body(prev_ref, g0_ref, g1_ref, a0_ref, a1_ref, root_ref,
                    bias_ref, y0_ref, y1_ref, out0_ref):
    x = jax.nn.relu(prev_ref[...] +
                    jnp.concatenate([g0_ref[...], g1_ref[...]], axis=1))
    y0_ref[...] = jnp.dot(x, a0_ref[...], preferred_element_type=f32)
    y1_ref[...] = jnp.dot(x, a1_ref[...], preferred_element_type=f32)
    out0_ref[...] = jnp.dot(x, root_ref[...], preferred_element_type=f32) + bias_ref[...]


_tc_layer2 = pl.pallas_call(
    _tc_layer2_body,
    grid=(NB,),
    in_specs=[
        pl.BlockSpec((BN, H), lambda i: (i, 0)),
        pl.BlockSpec((BN, 32), lambda i: (i, 0)),
        pl.BlockSpec((BN, 32), lambda i: (i, 0)),
        pl.BlockSpec((H, R * 32), lambda i: (0, 0)),
        pl.BlockSpec((H, R * 32), lambda i: (0, 0)),
        pl.BlockSpec((H, H), lambda i: (0, 0)),
        pl.BlockSpec((1, H), lambda i: (0, 0)),
    ],
    out_specs=[
        pl.BlockSpec((BN, R * 32), lambda i: (i, 0)),
        pl.BlockSpec((BN, R * 32), lambda i: (i, 0)),
        pl.BlockSpec((BN, H), lambda i: (i, 0)),
    ],
    out_shape=[
        jax.ShapeDtypeStruct((N, R * 32), f32),
        jax.ShapeDtypeStruct((N, R * 32), f32),
        jax.ShapeDtypeStruct((N, H), f32),
    ],
)


def _tc_head_body(prev_ref, g0_ref, g1_ref, b_ref, w1_ref, b1_ref, w2_ref,
                  b2_ref, logits_ref, psum, pcnt):
    i = pl.program_id(0)

    @pl.when(i == 0)
    def _init():
        psum[...] = jnp.zeros((G, H), f32)
        pcnt[...] = jnp.zeros((G, H), f32)

    x2 = jax.nn.relu(prev_ref[...] +
                     jnp.concatenate([g0_ref[...], g1_ref[...]], axis=1))
    oh = (b_ref[...] == lax.broadcasted_iota(i32, (1, G), 1)).astype(f32)
    psum[...] += lax.dot_general(oh, x2, (((0,), (0,)), ((), ())),
                                 preferred_element_type=f32)
    pcnt[...] += lax.dot_general(oh, jnp.ones((BN, H), f32),
                                 (((0,), (0,)), ((), ())),
                                 preferred_element_type=f32)

    @pl.when(i == NB - 1)
    def _fin():
        g = psum[...] / jnp.maximum(pcnt[...], 1.0)
        h = jax.nn.relu(jnp.dot(g, w1_ref[...], preferred_element_type=f32)
                        + b1_ref[...])
        logits_ref[...] = jnp.dot(h, w2_ref[...], preferred_element_type=f32) + b2_ref[...]


_tc_head = pl.pallas_call(
    _tc_head_body,
    grid=(NB,),
    in_specs=[
        pl.BlockSpec((BN, H), lambda i: (i, 0)),
        pl.BlockSpec((BN, 32), lambda i: (i, 0)),
        pl.BlockSpec((BN, 32), lambda i: (i, 0)),
        pl.BlockSpec((BN, 1), lambda i: (i, 0)),
        pl.BlockSpec((H, H), lambda i: (0, 0)),
        pl.BlockSpec((1, H), lambda i: (0, 0)),
        pl.BlockSpec((H, 128), lambda i: (0, 0)),
        pl.BlockSpec((1, 128), lambda i: (0, 0)),
    ],
    out_specs=pl.BlockSpec((G, 128), lambda i: (0, 0)),
    out_shape=jax.ShapeDtypeStruct((G, 128), f32),
    scratch_shapes=[
        pltpu.VMEM((G, H), f32),
        pltpu.VMEM((G, H), f32),
    ],
)


def kernel(emb, W, root, bias, W1, b1, W2, b2, node_type, edge_index,
           edge_type, batch):
    src = edge_index[0].astype(i32)
    dst = edge_index[1].astype(i32)
    et = edge_type.astype(i32)
    didx = dst * R + et
    gidx = src * R + et
    didx_p = jnp.concatenate(
        [didx, jnp.full((PADE,), N * R, i32)]).reshape(NROW_HBM, SUB)
    gidx_p = jnp.concatenate(
        [gidx, jnp.zeros((PADE,), i32)]).reshape(NROW_HBM, SUB)
    dst_p = jnp.concatenate(
        [dst, jnp.full((PADE,), N, i32)]).reshape(NROW_HBM, SUB)

    cnt0, cnt1 = _sc_counts(didx_p)
    inv = _tc_inv(cnt0.reshape(TBL // 128, 128),
                  cnt1.reshape(TBL // 128, 128)).reshape(TBL)
    w = _sc_weights(didx_p, inv)

    nt2 = node_type.reshape(N, 1).astype(i32)
    bat2 = batch.reshape(N, 1).astype(i32)
    W2p = jnp.zeros((H, 128), f32).at[:, :C].set(W2)
    b2p = jnp.zeros((1, 128), f32).at[0, :C].set(b2)
    A = [[W[l, :, :, cc * 32:(cc + 1) * 32].transpose(1, 0, 2).reshape(H, R * 32)
          for cc in range(2)] for l in range(2)]

    y0, y1, out0 = _tc_layer1(nt2, emb, A[0][0], A[0][1], root[0],
                              bias[0].reshape(1, H))
    agg = _sc_agg(y0.reshape(N * R, 32), y1.reshape(N * R, 32),
                  gidx_p, dst_p, w)
    y0b, y1b, out0b = _tc_layer2(out0, agg[0], agg[1], A[1][0], A[1][1],
                                 root[1], bias[1].reshape(1, H))
    agg2 = _sc_agg(y0b.reshape(N * R, 32), y1b.reshape(N * R, 32),
                   gidx_p, dst_p, w)
    logits_p = _tc_head(out0b, agg2[0], agg2[1], bat2, W1,
                        b1.reshape(1, H), W2p, b2p)
    return logits_p[:, :C]


# speedup vs baseline: 16.1163x; 1.3135x over previous
import jax
import jax.numpy as jnp
from jax import lax
from jax.experimental import pallas as pl
from jax.experimental.pallas import tpu as pltpu
from jax.experimental.pallas import tpu_sc as plsc

N = 50000
E = 800000
H = 64
R = 4
T = 8
G = 64
C = 2

NC = 2
NS = 16
SUB = 128
RPC = 4
K = RPC * SUB
EP = 50 * NC * NS * K
PADE = EP - E
NROW_HBM = EP // SUB
TBL = 200704
ZSL = TBL // 8
NROWS = N + 32
ZT = 10
ZROWS = N // ZT

BN = 1000
NB = N // BN

f32 = jnp.float32
i32 = jnp.int32

_mesh = plsc.VectorSubcoreMesh(core_axis_name="c", subcore_axis_name="s")


def _sc_counts_body(didx_hbm, cnt_out0, cnt_out1, cnt_sh, dix_v, zbuf, ones_v):
    c = lax.axis_index("c")
    s = lax.axis_index("s")
    wid = s * NC + c
    for i in range(250):
        zbuf[pl.ds(i * 16, 16)] = jnp.zeros((16,), f32)
    for i in range(8):
        ones_v[pl.ds(i * 16, 16)] = jnp.ones((16,), f32)

    @pl.when(s < 8)
    def _zero():
        zbase = s * ZSL
        for i in range(6):
            pltpu.sync_copy(zbuf, cnt_sh.at[pl.ds(zbase + i * 4000, 4000)])
        pltpu.sync_copy(zbuf.at[pl.ds(0, ZSL - 24000)],
                        cnt_sh.at[pl.ds(zbase + 24000, ZSL - 24000)])

    plsc.subcore_barrier()

    rpw = NROW_HBM // (NC * NS)

    def chunk(i, carry):
        row = wid * rpw + i * RPC
        pltpu.sync_copy(didx_hbm.at[pl.ds(row, RPC)], dix_v)
        for q in range(RPC):
            pltpu.sync_copy(ones_v, cnt_sh.at[dix_v.at[q]], add=True)
        return carry

    lax.fori_loop(0, rpw // RPC, chunk, 0)
    plsc.subcore_barrier()

    @pl.when((s < 8) & (c == 0))
    def _wb0():
        pltpu.sync_copy(cnt_sh.at[pl.ds(s * ZSL, ZSL)],
                        cnt_out0.at[pl.ds(s * ZSL, ZSL)])

    @pl.when((s < 8) & (c == 1))
    def _wb1():
        pltpu.sync_copy(cnt_sh.at[pl.ds(s * ZSL, ZSL)],
                        cnt_out1.at[pl.ds(s * ZSL, ZSL)])


_sc_counts = pl.kernel(
    _sc_counts_body,
    out_type=(jax.ShapeDtypeStruct((TBL,), f32),
              jax.ShapeDtypeStruct((TBL,), f32)),
    mesh=_mesh,
    scratch_types=[
        pltpu.VMEM_SHARED((TBL,), f32),
        pltpu.VMEM((RPC, SUB), i32),
        pltpu.VMEM((4000,), f32),
        pltpu.VMEM((SUB,), f32),
    ],
    compiler_params=pltpu.CompilerParams(use_tc_tiling_on_sc=False),
)


def _sc_weights_body(didx_hbm, inv_hbm, w_out, dix_v, w_v, sem):
    c = lax.axis_index("c")
    s = lax.axis_index("s")
    wid = s * NC + c
    rpw = NROW_HBM // (NC * NS)

    def chunk(i, carry):
        row = wid * rpw + i * RPC
        pltpu.sync_copy(didx_hbm.at[pl.ds(row, RPC)], dix_v)
        for q in range(RPC):
            pltpu.async_copy(inv_hbm.at[dix_v.at[q]],
                             w_v.at[pl.ds(q * SUB, SUB)], sem).wait()
        base = pl.multiple_of(row * SUB, K)
        pltpu.sync_copy(w_v, w_out.at[pl.ds(base, K)])
        return carry

    lax.fori_loop(0, rpw // RPC, chunk, 0)


_sc_weights = pl.kernel(
    _sc_weights_body,
    out_type=jax.ShapeDtypeStruct((EP,), f32),
    mesh=_mesh,
    scratch_types=[
        pltpu.VMEM((RPC, SUB), i32),
        pltpu.VMEM((K,), f32),
        pltpu.SemaphoreType.DMA,
    ],
    compiler_params=pltpu.CompilerParams(use_tc_tiling_on_sc=False),
)


def _sc_agg_body(y_hbm, gidx0_hbm, gidx1_hbm, dst_hbm, w_hbm, agg_out,
                 agg_sh, gix_v, dix_v, w_v, rows_v, zb2,
                 sem0, sem1, sem2, sem3):
    c = lax.axis_index("c")
    s = lax.axis_index("s")
    sems = [sem0, sem1, sem2, sem3]

    for i in range(125):
        zb2[i, pl.ds(0, 16)] = jnp.zeros((16,), f32)
        zb2[i, pl.ds(16, 16)] = jnp.zeros((16,), f32)

    @pl.when(s < ZT)
    def _zero():
        for i in range(ZROWS // 125):
            pltpu.sync_copy(zb2, agg_sh.at[pl.ds(s * ZROWS + i * 125, 125)])

    plsc.subcore_barrier()

    rpt = NROW_HBM // NS
    dn = lax.GatherDimensionNumbers(offset_dims=(), collapsed_slice_dims=(0,),
                                    start_index_map=(0,))

    def chunk(i, carry):
        row = s * rpt + i * RPC
        hd = pltpu.async_copy(dst_hbm.at[pl.ds(row, RPC)], dix_v, sem1)
        base = pl.multiple_of(row * SUB, K)
        hw = pltpu.async_copy(w_hbm.at[pl.ds(base, K)], w_v, sem2)

        @pl.when(c == 0)
        def _i0():
            pltpu.async_copy(gidx0_hbm.at[pl.ds(row, RPC)], gix_v, sem0).wait()

        @pl.when(c == 1)
        def _i1():
            pltpu.async_copy(gidx1_hbm.at[pl.ds(row, RPC)], gix_v, sem0).wait()

        hd.wait()
        hw.wait()
        handles = [
            pltpu.async_copy(y_hbm.at[gix_v.at[q]],
                             rows_v.at[pl.ds(q * SUB, SUB)], sems[q])
            for q in range(RPC)
        ]

        for q in range(RPC):
            handles[q].wait()

            def wgrp(j, carry2):
                wv = w_v[pl.ds(q * SUB + j * 16, 16)]
                for t in range(16):
                    e = q * SUB + j * 16 + t
                    wb = lax.gather(wv, jnp.full((16, 1), t, i32), dn, (1,),
                                    mode=lax.GatherScatterMode.PROMISE_IN_BOUNDS)
                    r0 = rows_v[e, pl.ds(0, 16)]
                    rows_v[e, pl.ds(0, 16)] = r0 * wb
                    r1 = rows_v[e, pl.ds(16, 16)]
                    rows_v[e, pl.ds(16, 16)] = r1 * wb
                return carry2

            lax.fori_loop(0, SUB // 16, wgrp, 0)
            pltpu.sync_copy(rows_v.at[pl.ds(q * SUB, SUB)],
                            agg_sh.at[dix_v.at[q]], add=True)
        return carry

    lax.fori_loop(0, rpt // RPC, chunk, 0)
    plsc.subcore_barrier()

    @pl.when(s < ZT)
    def _wb():
        pltpu.sync_copy(agg_sh.at[pl.ds(s * ZROWS, ZROWS)],
                        agg_out.at[c, pl.ds(s * ZROWS, ZROWS)])


_sc_agg = pl.kernel(
    _sc_agg_body,
    out_type=jax.ShapeDtypeStruct((NC, N, 32), f32),
    mesh=_mesh,
    scratch_types=[
        pltpu.VMEM_SHARED((NROWS, 32), f32),
        pltpu.VMEM((RPC, SUB), i32),
        pltpu.VMEM((RPC, SUB), i32),
        pltpu.VMEM((K,), f32),
        pltpu.VMEM((K, 32), f32),
        pltpu.VMEM((125, 32), f32),
        pltpu.SemaphoreType.DMA,
        pltpu.SemaphoreType.DMA,
        pltpu.SemaphoreType.DMA,
        pltpu.SemaphoreType.DMA,
    ],
    compiler_params=pltpu.CompilerParams(use_tc_tiling_on_sc=False),
)


def _tc_inv_body(c0_ref, c1_ref, inv_ref):
    inv_ref[...] = 1.0 / jnp.maximum(c0_ref[...] + c1_ref[...], 1.0)


_tc_inv = pl.pallas_call(
    _tc_inv_body,
    grid=(1,),
    in_specs=[
        pl.BlockSpec((TBL // 128, 128), lambda i: (0, 0)),
        pl.BlockSpec((TBL // 128, 128), lambda i: (0, 0)),
    ],
    out_specs=pl.BlockSpec((TBL // 128, 128), lambda i: (0, 0)),
    out_shape=jax.ShapeDtypeStruct((TBL // 128, 128), f32),
)


def _tc_layer1_body(nt_ref, emb_ref, a0_ref, a1_ref, root_ref, bias_ref,
                    y_ref, out0_ref):
    nt = nt_ref[...]
    oh = (nt == lax.broadcasted_iota(i32, (1, T), 1)).astype(f32)
    x = jnp.dot(oh, emb_ref[...], preferred_element_type=f32)
    y_ref[0] = jnp.dot(x, a0_ref[...], preferred_element_type=f32)
    y_ref[1] = jnp.dot(x, a1_ref[...], preferred_element_type=f32)
    out0_ref[...] = jnp.dot(x, root_ref[...], preferred_element_type=f32) + bias_ref[...]


_tc_layer1 = pl.pallas_call(
    _tc_layer1_body,
    grid=(NB,),
    in_specs=[
        pl.BlockSpec((BN, 1), lambda i: (i, 0)),
        pl.BlockSpec((T, H), lambda i: (0, 0)),
        pl.BlockSpec((H, R * 32), lambda i: (0, 0)),
        pl.BlockSpec((H, R * 32), lambda i: (0, 0)),
        pl.BlockSpec((H, H), lambda i: (0, 0)),
        pl.BlockSpec((1, H), lambda i: (0, 0)),
    ],
    out_specs=[
        pl.BlockSpec((2, BN, R * 32), lambda i: (0, i, 0)),
        pl.BlockSpec((BN, H), lambda i: (i, 0)),
    ],
    out_shape=[
        jax.ShapeDtypeStruct((2, N, R * 32), f32),
        jax.ShapeDtypeStruct((N, H), f32),
    ],
)


def _tc_layer2_body(prev_ref, g0_ref, g1_ref, a0_ref, a1_ref, root_ref,
                    bias_ref, y_ref, out0_ref):
    x = jax.nn.relu(prev_ref[...] +
                    jnp.concatenate([g0_ref[...], g1_ref[...]], axis=1))
    y_ref[0] = jnp.dot(x, a0_ref[...], preferred_element_type=f32)
    y_ref[1] = jnp.dot(x, a1_ref[...], preferred_element_type=f32)
    out0_ref[...] = jnp.dot(x, root_ref[...], preferred_element_type=f32) + bias_ref[...]


_tc_layer2 = pl.pallas_call(
    _tc_layer2_body,
    grid=(NB,),
    in_specs=[
        pl.BlockSpec((BN, H), lambda i: (i, 0)),
        pl.BlockSpec((BN, 32), lambda i: (i, 0)),
        pl.BlockSpec((BN, 32), lambda i: (i, 0)),
        pl.BlockSpec((H, R * 32), lambda i: (0, 0)),
        pl.BlockSpec((H, R * 32), lambda i: (0, 0)),
        pl.BlockSpec((H, H), lambda i: (0, 0)),
        pl.BlockSpec((1, H), lambda i: (0, 0)),
    ],
    out_specs=[
        pl.BlockSpec((2, BN, R * 32), lambda i: (0, i, 0)),
        pl.BlockSpec((BN, H), lambda i: (i, 0)),
    ],
    out_shape=[
        jax.ShapeDtypeStruct((2, N, R * 32), f32),
        jax.ShapeDtypeStruct((N, H), f32),
    ],
)


def _tc_head_body(prev_ref, g0_ref, g1_ref, b_ref, w1_ref, b1_ref, w2_ref,
                  b2_ref, logits_ref, psum, pcnt):
    i = pl.program_id(0)

    @pl.when(i == 0)
    def _init():
        psum[...] = jnp.zeros((G, H), f32)
        pcnt[...] = jnp.zeros((G, H), f32)

    x2 = jax.nn.relu(prev_ref[...] +
                     jnp.concatenate([g0_ref[...], g1_ref[...]], axis=1))
    oh = (b_ref[...] == lax.broadcasted_iota(i32, (1, G), 1)).astype(f32)
    psum[...] += lax.dot_general(oh, x2, (((0,), (0,)), ((), ())),
                                 preferred_element_type=f32)
    pcnt[...] += lax.dot_general(oh, jnp.ones((BN, H), f32),
                                 (((0,), (0,)), ((), ())),
                                 preferred_element_type=f32)

    @pl.when(i == NB - 1)
    def _fin():
        g = psum[...] / jnp.maximum(pcnt[...], 1.0)
        h = jax.nn.relu(jnp.dot(g, w1_ref[...], preferred_element_type=f32)
                        + b1_ref[...])
        logits_ref[...] = jnp.dot(h, w2_ref[...], preferred_element_type=f32) + b2_ref[...]


_tc_head = pl.pallas_call(
    _tc_head_body,
    grid=(NB,),
    in_specs=[
        pl.BlockSpec((BN, H), lambda i: (i, 0)),
        pl.BlockSpec((BN, 32), lambda i: (i, 0)),
        pl.BlockSpec((BN, 32), lambda i: (i, 0)),
        pl.BlockSpec((BN, 1), lambda i: (i, 0)),
        pl.BlockSpec((H, H), lambda i: (0, 0)),
        pl.BlockSpec((1, H), lambda i: (0, 0)),
        pl.BlockSpec((H, 128), lambda i: (0, 0)),
        pl.BlockSpec((1, 128), lambda i: (0, 0)),
    ],
    out_specs=pl.BlockSpec((G, 128), lambda i: (0, 0)),
    out_shape=jax.ShapeDtypeStruct((G, 128), f32),
    scratch_shapes=[
        pltpu.VMEM((G, H), f32),
        pltpu.VMEM((G, H), f32),
    ],
)


def kernel(emb, W, root, bias, W1, b1, W2, b2, node_type, edge_index,
           edge_type, batch):
    src = edge_index[0].astype(i32)
    dst = edge_index[1].astype(i32)
    et = edge_type.astype(i32)
    didx = dst * R + et
    gidx = src * R + et
    didx_p = jnp.concatenate(
        [didx, jnp.full((PADE,), N * R, i32)]).reshape(NROW_HBM, SUB)
    gidx_p = jnp.concatenate(
        [gidx, jnp.zeros((PADE,), i32)]).reshape(NROW_HBM, SUB)
    gidx_p1 = gidx_p + N * R
    dst_p = jnp.concatenate(
        [dst, jnp.full((PADE,), N, i32)]).reshape(NROW_HBM, SUB)

    cnt0, cnt1 = _sc_counts(didx_p)
    inv = _tc_inv(cnt0.reshape(TBL // 128, 128),
                  cnt1.reshape(TBL // 128, 128)).reshape(TBL)
    w = _sc_weights(didx_p, inv)

    nt2 = node_type.reshape(N, 1).astype(i32)
    bat2 = batch.reshape(N, 1).astype(i32)
    W2p = jnp.zeros((H, 128), f32).at[:, :C].set(W2)
    b2p = jnp.zeros((1, 128), f32).at[0, :C].set(b2)
    A = [[W[l, :, :, cc * 32:(cc + 1) * 32].transpose(1, 0, 2).reshape(H, R * 32)
          for cc in range(2)] for l in range(2)]

    y, out0 = _tc_layer1(nt2, emb, A[0][0], A[0][1], root[0],
                         bias[0].reshape(1, H))
    agg = _sc_agg(y.reshape(2 * N * R, 32), gidx_p, gidx_p1, dst_p, w)
    yb, out0b = _tc_layer2(out0, agg[0], agg[1], A[1][0], A[1][1],
                           root[1], bias[1].reshape(1, H))
    agg2 = _sc_agg(yb.reshape(2 * N * R, 32), gidx_p, gidx_p1, dst_p, w)
    logits_p = _tc_head(out0b, agg2[0], agg2[1], bat2, W1,
                        b1.reshape(1, H), W2p, b2p)
    return logits_p[:, :C]


# speedup vs baseline: 18.6691x; 1.1584x over previous
import jax
import jax.numpy as jnp
from jax import lax
from jax.experimental import pallas as pl
from jax.experimental.pallas import tpu as pltpu
from jax.experimental.pallas import tpu_sc as plsc

N = 50000
E = 800000
H = 64
R = 4
T = 8
G = 64
C = 2

NC = 2
NS = 16
SUB = 128
RPC = 4
K = RPC * SUB
EP = 50 * NC * NS * K
PADE = EP - E
NROW_HBM = EP // SUB
NROW_X = NROW_HBM + 2 * RPC
EPX = NROW_X * SUB
TBL = 200704
ZSL = TBL // 8
NROWS = N + 32
ZT = 10
ZROWS = N // ZT

BN = 1000
NB = N // BN

f32 = jnp.float32
i32 = jnp.int32

_mesh = plsc.VectorSubcoreMesh(core_axis_name="c", subcore_axis_name="s")


def _sc_counts_body(didx_hbm, cnt_out0, cnt_out1, cnt_sh, dix_v, zbuf, ones_v):
    c = lax.axis_index("c")
    s = lax.axis_index("s")
    wid = s * NC + c
    for i in range(250):
        zbuf[pl.ds(i * 16, 16)] = jnp.zeros((16,), f32)
    for i in range(8):
        ones_v[pl.ds(i * 16, 16)] = jnp.ones((16,), f32)

    @pl.when(s < 8)
    def _zero():
        zbase = s * ZSL
        for i in range(6):
            pltpu.sync_copy(zbuf, cnt_sh.at[pl.ds(zbase + i * 4000, 4000)])
        pltpu.sync_copy(zbuf.at[pl.ds(0, ZSL - 24000)],
                        cnt_sh.at[pl.ds(zbase + 24000, ZSL - 24000)])

    plsc.subcore_barrier()

    rpw = NROW_HBM // (NC * NS)

    def chunk(i, carry):
        row = wid * rpw + i * RPC
        pltpu.sync_copy(didx_hbm.at[pl.ds(row, RPC)], dix_v)
        for q in range(RPC):
            pltpu.sync_copy(ones_v, cnt_sh.at[dix_v.at[q]], add=True)
        return carry

    lax.fori_loop(0, rpw // RPC, chunk, 0)
    plsc.subcore_barrier()

    @pl.when((s < 8) & (c == 0))
    def _wb0():
        pltpu.sync_copy(cnt_sh.at[pl.ds(s * ZSL, ZSL)],
                        cnt_out0.at[pl.ds(s * ZSL, ZSL)])

    @pl.when((s < 8) & (c == 1))
    def _wb1():
        pltpu.sync_copy(cnt_sh.at[pl.ds(s * ZSL, ZSL)],
                        cnt_out1.at[pl.ds(s * ZSL, ZSL)])


_sc_counts = pl.kernel(
    _sc_counts_body,
    out_type=(jax.ShapeDtypeStruct((TBL,), f32),
              jax.ShapeDtypeStruct((TBL,), f32)),
    mesh=_mesh,
    scratch_types=[
        pltpu.VMEM_SHARED((TBL,), f32),
        pltpu.VMEM((RPC, SUB), i32),
        pltpu.VMEM((4000,), f32),
        pltpu.VMEM((SUB,), f32),
    ],
    compiler_params=pltpu.CompilerParams(use_tc_tiling_on_sc=False),
)


def _sc_weights_body(didx_hbm, inv_hbm, w_out, dix_v, w_v, sem):
    c = lax.axis_index("c")
    s = lax.axis_index("s")
    wid = s * NC + c
    rpw = NROW_HBM // (NC * NS)

    def chunk(i, carry):
        row = wid * rpw + i * RPC
        pltpu.sync_copy(didx_hbm.at[pl.ds(row, RPC)], dix_v)
        for q in range(RPC):
            pltpu.async_copy(inv_hbm.at[dix_v.at[q]],
                             w_v.at[pl.ds(q * SUB, SUB)], sem).wait()
        base = pl.multiple_of(row * SUB, K)
        pltpu.sync_copy(w_v, w_out.at[pl.ds(base, K)])
        return carry

    lax.fori_loop(0, rpw // RPC, chunk, 0)


_sc_weights = pl.kernel(
    _sc_weights_body,
    out_type=jax.ShapeDtypeStruct((EP,), f32),
    mesh=_mesh,
    scratch_types=[
        pltpu.VMEM((RPC, SUB), i32),
        pltpu.VMEM((K,), f32),
        pltpu.SemaphoreType.DMA,
    ],
    compiler_params=pltpu.CompilerParams(use_tc_tiling_on_sc=False),
)


def _sc_agg_body(y_hbm, gidx0_hbm, gidx1_hbm, dst_hbm, w_hbm, agg_out,
                 agg_sh, gixA, dixA, wA, gixB, dixB, wB, rows_v, zb2,
                 sg0, sg1, sg2, sg3, sa0, sa1, sa2, sb0, sb1, sb2):
    c = lax.axis_index("c")
    s = lax.axis_index("s")
    sg = [sg0, sg1, sg2, sg3]
    bufA = (gixA, dixA, wA, sa0, sa1, sa2)
    bufB = (gixB, dixB, wB, sb0, sb1, sb2)

    for i in range(125):
        zb2[i, pl.ds(0, 16)] = jnp.zeros((16,), f32)
        zb2[i, pl.ds(16, 16)] = jnp.zeros((16,), f32)

    @pl.when(s < ZT)
    def _zero():
        for i in range(ZROWS // 125):
            pltpu.sync_copy(zb2, agg_sh.at[pl.ds(s * ZROWS + i * 125, 125)])

    plsc.subcore_barrier()

    rpt = NROW_HBM // NS
    NCHUNK = rpt // RPC
    dn = lax.GatherDimensionNumbers(offset_dims=(), collapsed_slice_dims=(0,),
                                    start_index_map=(0,))
    base0 = s * rpt

    def fetch_idx(i, buf):
        gix, dix, wv, s_g, s_d, s_w = buf
        row = base0 + i * RPC

        @pl.when(c == 0)
        def _i0():
            pltpu.async_copy(gidx0_hbm.at[pl.ds(row, RPC)], gix, s_g)

        @pl.when(c == 1)
        def _i1():
            pltpu.async_copy(gidx1_hbm.at[pl.ds(row, RPC)], gix, s_g)

        pltpu.async_copy(dst_hbm.at[pl.ds(row, RPC)], dix, s_d)
        base = pl.multiple_of(row * SUB, K)
        pltpu.async_copy(w_hbm.at[pl.ds(base, K)], wv, s_w)

    def wait_gix(buf):
        gix, _, _, s_g, _, _ = buf
        pltpu.make_async_copy(gidx0_hbm.at[pl.ds(0, RPC)], gix, s_g).wait()

    def wait_dw(buf):
        _, dix, wv, _, s_d, s_w = buf
        pltpu.make_async_copy(dst_hbm.at[pl.ds(0, RPC)], dix, s_d).wait()
        pltpu.make_async_copy(w_hbm.at[pl.ds(0, K)], wv, s_w).wait()

    def gather_slot(buf, q):
        gix = buf[0]
        pltpu.async_copy(y_hbm.at[gix.at[q]],
                         rows_v.at[pl.ds(q * SUB, SUB)], sg[q])

    def wait_slot(buf, q):
        gix = buf[0]
        pltpu.make_async_copy(y_hbm.at[gix.at[q]],
                              rows_v.at[pl.ds(q * SUB, SUB)], sg[q]).wait()

    def drain(cur, nxt):
        w_c = cur[2]
        dix_c = cur[1]
        for q in range(RPC):
            wait_slot(cur, q)

            def wgrp(j, carry2):
                wv = w_c[pl.ds(q * SUB + j * 16, 16)]
                for t in range(16):
                    e = q * SUB + j * 16 + t
                    wb = lax.gather(wv, jnp.full((16, 1), t, i32), dn, (1,),
                                    mode=lax.GatherScatterMode.PROMISE_IN_BOUNDS)
                    r0 = rows_v[e, pl.ds(0, 16)]
                    rows_v[e, pl.ds(0, 16)] = r0 * wb
                    r1 = rows_v[e, pl.ds(16, 16)]
                    rows_v[e, pl.ds(16, 16)] = r1 * wb
                return carry2

            lax.fori_loop(0, SUB // 16, wgrp, 0)
            pltpu.sync_copy(rows_v.at[pl.ds(q * SUB, SUB)],
                            agg_sh.at[dix_c.at[q]], add=True)
            if q == 0:
                wait_gix(nxt)
            gather_slot(nxt, q)

    fetch_idx(0, bufA)
    wait_gix(bufA)
    for q in range(RPC):
        gather_slot(bufA, q)
    fetch_idx(1, bufB)

    def two_chunks(j, carry):
        i = j * 2
        wait_dw(bufA)
        drain(bufA, bufB)
        fetch_idx(i + 2, bufA)
        wait_dw(bufB)
        drain(bufB, bufA)
        fetch_idx(i + 3, bufB)
        return carry

    lax.fori_loop(0, NCHUNK // 2, two_chunks, 0)

    wait_dw(bufA)
    wait_gix(bufB)
    wait_dw(bufB)
    for q in range(RPC):
        wait_slot(bufA, q)
    plsc.subcore_barrier()

    @pl.when(s < ZT)
    def _wb():
        pltpu.sync_copy(agg_sh.at[pl.ds(s * ZROWS, ZROWS)],
                        agg_out.at[c, pl.ds(s * ZROWS, ZROWS)])


_sc_agg = pl.kernel(
    _sc_agg_body,
    out_type=jax.ShapeDtypeStruct((NC, N, 32), f32),
    mesh=_mesh,
    scratch_types=[
        pltpu.VMEM_SHARED((NROWS, 32), f32),
        pltpu.VMEM((RPC, SUB), i32),
        pltpu.VMEM((RPC, SUB), i32),
        pltpu.VMEM((K,), f32),
        pltpu.VMEM((RPC, SUB), i32),
        pltpu.VMEM((RPC, SUB), i32),
        pltpu.VMEM((K,), f32),
        pltpu.VMEM((K, 32), f32),
        pltpu.VMEM((125, 32), f32),
        pltpu.SemaphoreType.DMA,
        pltpu.SemaphoreType.DMA,
        pltpu.SemaphoreType.DMA,
        pltpu.SemaphoreType.DMA,
        pltpu.SemaphoreType.DMA,
        pltpu.SemaphoreType.DMA,
        pltpu.SemaphoreType.DMA,
        pltpu.SemaphoreType.DMA,
        pltpu.SemaphoreType.DMA,
        pltpu.SemaphoreType.DMA,
    ],
    compiler_params=pltpu.CompilerParams(use_tc_tiling_on_sc=False),
)


def _tc_inv_body(c0_ref, c1_ref, inv_ref):
    inv_ref[...] = 1.0 / jnp.maximum(c0_ref[...] + c1_ref[...], 1.0)


_tc_inv = pl.pallas_call(
    _tc_inv_body,
    grid=(1,),
    in_specs=[
        pl.BlockSpec((TBL // 128, 128), lambda i: (0, 0)),
        pl.BlockSpec((TBL // 128, 128), lambda i: (0, 0)),
    ],
    out_specs=pl.BlockSpec((TBL // 128, 128), lambda i: (0, 0)),
    out_shape=jax.ShapeDtypeStruct((TBL // 128, 128), f32),
)


def _tc_layer1_body(nt_ref, emb_ref, a0_ref, a1_ref, root_ref, bias_ref,
                    y_ref, out0_ref):
    nt = nt_ref[...]
    oh = (nt == lax.broadcasted_iota(i32, (1, T), 1)).astype(f32)
    x = jnp.dot(oh, emb_ref[...], preferred_element_type=f32)
    y_ref[0] = jnp.dot(x, a0_ref[...], preferred_element_type=f32)
    y_ref[1] = jnp.dot(x, a1_ref[...], preferred_element_type=f32)
    out0_ref[...] = jnp.dot(x, root_ref[...], preferred_element_type=f32) + bias_ref[...]


_tc_layer1 = pl.pallas_call(
    _tc_layer1_body,
    grid=(NB,),
    in_specs=[
        pl.BlockSpec((BN, 1), lambda i: (i, 0)),
        pl.BlockSpec((T, H), lambda i: (0, 0)),
        pl.BlockSpec((H, R * 32), lambda i: (0, 0)),
        pl.BlockSpec((H, R * 32), lambda i: (0, 0)),
        pl.BlockSpec((H, H), lambda i: (0, 0)),
        pl.BlockSpec((1, H), lambda i: (0, 0)),
    ],
    out_specs=[
        pl.BlockSpec((2, BN, R * 32), lambda i: (0, i, 0)),
        pl.BlockSpec((BN, H), lambda i: (i, 0)),
    ],
    out_shape=[
        jax.ShapeDtypeStruct((2, N, R * 32), f32),
        jax.ShapeDtypeStruct((N, H), f32),
    ],
)


def _tc_layer2_body(prev_ref, g0_ref, g1_ref, a0_ref, a1_ref, root_ref,
                    bias_ref, y_ref, out0_ref):
    x = jax.nn.relu(prev_ref[...] +
                    jnp.concatenate([g0_ref[...], g1_ref[...]], axis=1))
    y_ref[0] = jnp.dot(x, a0_ref[...], preferred_element_type=f32)
    y_ref[1] = jnp.dot(x, a1_ref[...], preferred_element_type=f32)
    out0_ref[...] = jnp.dot(x, root_ref[...], preferred_element_type=f32) + bias_ref[...]


_tc_layer2 = pl.pallas_call(
    _tc_layer2_body,
    grid=(NB,),
    in_specs=[
        pl.BlockSpec((BN, H), lambda i: (i, 0)),
        pl.BlockSpec((BN, 32), lambda i: (i, 0)),
        pl.BlockSpec((BN, 32), lambda i: (i, 0)),
        pl.BlockSpec((H, R * 32), lambda i: (0, 0)),
        pl.BlockSpec((H, R * 32), lambda i: (0, 0)),
        pl.BlockSpec((H, H), lambda i: (0, 0)),
        pl.BlockSpec((1, H), lambda i: (0, 0)),
    ],
    out_specs=[
        pl.BlockSpec((2, BN, R * 32), lambda i: (0, i, 0)),
        pl.BlockSpec((BN, H), lambda i: (i, 0)),
    ],
    out_shape=[
        jax.ShapeDtypeStruct((2, N, R * 32), f32),
        jax.ShapeDtypeStruct((N, H), f32),
    ],
)


def _tc_head_body(prev_ref, g0_ref, g1_ref, b_ref, w1_ref, b1_ref, w2_ref,
                  b2_ref, logits_ref, psum, pcnt):
    i = pl.program_id(0)

    @pl.when(i == 0)
    def _init():
        psum[...] = jnp.zeros((G, H), f32)
        pcnt[...] = jnp.zeros((G, H), f32)

    x2 = jax.nn.relu(prev_ref[...] +
                     jnp.concatenate([g0_ref[...], g1_ref[...]], axis=1))
    oh = (b_ref[...] == lax.broadcasted_iota(i32, (1, G), 1)).astype(f32)
    psum[...] += lax.dot_general(oh, x2, (((0,), (0,)), ((), ())),
                                 preferred_element_type=f32)
    pcnt[...] += lax.dot_general(oh, jnp.ones((BN, H), f32),
                                 (((0,), (0,)), ((), ())),
                                 preferred_element_type=f32)

    @pl.when(i == NB - 1)
    def _fin():
        g = psum[...] / jnp.maximum(pcnt[...], 1.0)
        h = jax.nn.relu(jnp.dot(g, w1_ref[...], preferred_element_type=f32)
                        + b1_ref[...])
        logits_ref[...] = jnp.dot(h, w2_ref[...], preferred_element_type=f32) + b2_ref[...]


_tc_head = pl.pallas_call(
    _tc_head_body,
    grid=(NB,),
    in_specs=[
        pl.BlockSpec((BN, H), lambda i: (i, 0)),
        pl.BlockSpec((BN, 32), lambda i: (i, 0)),
        pl.BlockSpec((BN, 32), lambda i: (i, 0)),
        pl.BlockSpec((BN, 1), lambda i: (i, 0)),
        pl.BlockSpec((H, H), lambda i: (0, 0)),
        pl.BlockSpec((1, H), lambda i: (0, 0)),
        pl.BlockSpec((H, 128), lambda i: (0, 0)),
        pl.BlockSpec((1, 128), lambda i: (0, 0)),
    ],
    out_specs=pl.BlockSpec((G, 128), lambda i: (0, 0)),
    out_shape=jax.ShapeDtypeStruct((G, 128), f32),
    scratch_shapes=[
        pltpu.VMEM((G, H), f32),
        pltpu.VMEM((G, H), f32),
    ],
)


def kernel(emb, W, root, bias, W1, b1, W2, b2, node_type, edge_index,
           edge_type, batch):
    src = edge_index[0].astype(i32)
    dst = edge_index[1].astype(i32)
    et = edge_type.astype(i32)
    didx = dst * R + et
    gidx = src * R + et
    didx_p = jnp.concatenate(
        [didx, jnp.full((PADE,), N * R, i32)]).reshape(NROW_HBM, SUB)
    gidx_p = jnp.concatenate(
        [gidx, jnp.zeros((EPX - E,), i32)]).reshape(NROW_X, SUB)
    gidx_p1 = gidx_p + N * R
    dst_p = jnp.concatenate(
        [dst, jnp.full((EPX - E,), N, i32)]).reshape(NROW_X, SUB)

    cnt0, cnt1 = _sc_counts(didx_p)
    inv = _tc_inv(cnt0.reshape(TBL // 128, 128),
                  cnt1.reshape(TBL // 128, 128)).reshape(TBL)
    w = _sc_weights(didx_p, inv)
    w = jnp.concatenate([w, jnp.zeros((EPX - EP,), f32)])

    nt2 = node_type.reshape(N, 1).astype(i32)
    bat2 = batch.reshape(N, 1).astype(i32)
    W2p = jnp.zeros((H, 128), f32).at[:, :C].set(W2)
    b2p = jnp.zeros((1, 128), f32).at[0, :C].set(b2)
    A = [[W[l, :, :, cc * 32:(cc + 1) * 32].transpose(1, 0, 2).reshape(H, R * 32)
          for cc in range(2)] for l in range(2)]

    y, out0 = _tc_layer1(nt2, emb, A[0][0], A[0][1], root[0],
                         bias[0].reshape(1, H))
    agg = _sc_agg(y.reshape(2 * N * R, 32), gidx_p, gidx_p1, dst_p, w)
    yb, out0b = _tc_layer2(out0, agg[0], agg[1], A[1][0], A[1][1],
                           root[1], bias[1].reshape(1, H))
    agg2 = _sc_agg(yb.reshape(2 * N * R, 32), gidx_p, gidx_p1, dst_p, w)
    logits_p = _tc_head(out0b, agg2[0], agg2[1], bat2, W1,
                        b1.reshape(1, H), W2p, b2p)
    return logits_p[:, :C]


# speedup vs baseline: 18.9114x; 1.0130x over previous
import jax
import jax.numpy as jnp
from jax import lax
from jax.experimental import pallas as pl
from jax.experimental.pallas import tpu as pltpu
from jax.experimental.pallas import tpu_sc as plsc

N = 50000
E = 800000
H = 64
R = 4
T = 8
G = 64
C = 2

NC = 2
NS = 16
SUB = 128
RPC = 4
K = RPC * SUB
EP = 50 * NC * NS * K
PADE = EP - E
NROW_HBM = EP // SUB
NROW_X = NROW_HBM + 2 * RPC
EPX = NROW_X * SUB
TBL = 200704
ZSL = TBL // 8
NROWS = N + 32
ZT = 10
ZROWS = N // ZT

BN = 1000
NB = N // BN

f32 = jnp.float32
i32 = jnp.int32

_mesh = plsc.VectorSubcoreMesh(core_axis_name="c", subcore_axis_name="s")


def _sc_counts_body(didx_hbm, cnt_out0, cnt_out1, cnt_sh, dixA, dixB, zbuf,
                    ones_v, sfA, sfB, ssA, ssB, sz):
    c = lax.axis_index("c")
    s = lax.axis_index("s")
    wid = s * NC + c
    for i in range(250):
        zbuf[pl.ds(i * 16, 16)] = jnp.zeros((16,), f32)
    for i in range(8):
        ones_v[pl.ds(i * 16, 16)] = jnp.ones((16,), f32)

    @pl.when(s < 8)
    def _zero():
        zbase = s * ZSL
        for i in range(6):
            pltpu.async_copy(zbuf, cnt_sh.at[pl.ds(zbase + i * 4000, 4000)], sz)
        pltpu.async_copy(zbuf.at[pl.ds(0, ZSL - 24000)],
                         cnt_sh.at[pl.ds(zbase + 24000, ZSL - 24000)], sz)
        for i in range(6):
            pltpu.make_async_copy(zbuf, cnt_sh.at[pl.ds(zbase, 4000)], sz).wait()
        pltpu.make_async_copy(zbuf.at[pl.ds(0, ZSL - 24000)],
                              cnt_sh.at[pl.ds(zbase, ZSL - 24000)], sz).wait()

    plsc.subcore_barrier()

    rpw = NROW_HBM // (NC * NS)
    NCH = rpw // RPC
    base0 = wid * rpw

    def fetch(i, dix, sf):
        pltpu.async_copy(didx_hbm.at[pl.ds(base0 + i * RPC, RPC)], dix, sf)

    def wait_fetch(dix, sf):
        pltpu.make_async_copy(didx_hbm.at[pl.ds(0, RPC)], dix, sf).wait()

    def scatters(dix, ss):
        for q in range(RPC):
            pltpu.async_copy(ones_v, cnt_sh.at[dix.at[q]], ss, add=True)

    def wait_scatters(dix, ss):
        for q in range(RPC):
            pltpu.make_async_copy(ones_v, cnt_sh.at[dix.at[q]], ss).wait()

    fetch(0, dixA, sfA)
    fetch(1, dixB, sfB)
    wait_fetch(dixA, sfA)
    scatters(dixA, ssA)

    def two_chunks(j, carry):
        i = j * 2
        wait_fetch(dixB, sfB)
        scatters(dixB, ssB)
        wait_scatters(dixA, ssA)
        fetch(i + 2, dixA, sfA)
        wait_fetch(dixA, sfA)
        scatters(dixA, ssA)
        wait_scatters(dixB, ssB)
        fetch(i + 3, dixB, sfB)
        return carry

    lax.fori_loop(0, NCH // 2 - 1, two_chunks, 0)
    wait_fetch(dixB, sfB)
    scatters(dixB, ssB)
    wait_scatters(dixA, ssA)
    wait_scatters(dixB, ssB)
    plsc.subcore_barrier()

    @pl.when((s < 8) & (c == 0))
    def _wb0():
        pltpu.sync_copy(cnt_sh.at[pl.ds(s * ZSL, ZSL)],
                        cnt_out0.at[pl.ds(s * ZSL, ZSL)])

    @pl.when((s < 8) & (c == 1))
    def _wb1():
        pltpu.sync_copy(cnt_sh.at[pl.ds(s * ZSL, ZSL)],
                        cnt_out1.at[pl.ds(s * ZSL, ZSL)])


_sc_counts = pl.kernel(
    _sc_counts_body,
    out_type=(jax.ShapeDtypeStruct((TBL,), f32),
              jax.ShapeDtypeStruct((TBL,), f32)),
    mesh=_mesh,
    scratch_types=[
        pltpu.VMEM_SHARED((TBL,), f32),
        pltpu.VMEM((RPC, SUB), i32),
        pltpu.VMEM((RPC, SUB), i32),
        pltpu.VMEM((4000,), f32),
        pltpu.VMEM((SUB,), f32),
        pltpu.SemaphoreType.DMA,
        pltpu.SemaphoreType.DMA,
        pltpu.SemaphoreType.DMA,
        pltpu.SemaphoreType.DMA,
        pltpu.SemaphoreType.DMA,
    ],
    compiler_params=pltpu.CompilerParams(use_tc_tiling_on_sc=False),
)


def _sc_weights_body(didx_hbm, inv_hbm, w_out, dix_v, w_v, sem):
    c = lax.axis_index("c")
    s = lax.axis_index("s")
    wid = s * NC + c
    rpw = NROW_HBM // (NC * NS)

    def chunk(i, carry):
        row = wid * rpw + i * RPC
        pltpu.sync_copy(didx_hbm.at[pl.ds(row, RPC)], dix_v)
        for q in range(RPC):
            pltpu.async_copy(inv_hbm.at[dix_v.at[q]],
                             w_v.at[pl.ds(q * SUB, SUB)], sem).wait()
        base = pl.multiple_of(row * SUB, K)
        pltpu.sync_copy(w_v, w_out.at[pl.ds(base, K)])
        return carry

    lax.fori_loop(0, rpw // RPC, chunk, 0)


_sc_weights = pl.kernel(
    _sc_weights_body,
    out_type=jax.ShapeDtypeStruct((EP,), f32),
    mesh=_mesh,
    scratch_types=[
        pltpu.VMEM((RPC, SUB), i32),
        pltpu.VMEM((K,), f32),
        pltpu.SemaphoreType.DMA,
    ],
    compiler_params=pltpu.CompilerParams(use_tc_tiling_on_sc=False),
)


def _sc_agg_body(y_hbm, gidx0_hbm, gidx1_hbm, dst_hbm, w_hbm, agg_out,
                 agg_sh, gixA, dixA, wA, gixB, dixB, wB, rows_v, zb2,
                 sg0, sg1, sg2, sg3, sa0, sa1, sa2, sb0, sb1, sb2):
    c = lax.axis_index("c")
    s = lax.axis_index("s")
    sg = [sg0, sg1, sg2, sg3]
    bufA = (gixA, dixA, wA, sa0, sa1, sa2)
    bufB = (gixB, dixB, wB, sb0, sb1, sb2)

    for i in range(125):
        zb2[i, pl.ds(0, 16)] = jnp.zeros((16,), f32)
        zb2[i, pl.ds(16, 16)] = jnp.zeros((16,), f32)

    @pl.when(s < ZT)
    def _zero():
        for i in range(ZROWS // 125):
            pltpu.sync_copy(zb2, agg_sh.at[pl.ds(s * ZROWS + i * 125, 125)])

    plsc.subcore_barrier()

    rpt = NROW_HBM // NS
    NCHUNK = rpt // RPC
    dn = lax.GatherDimensionNumbers(offset_dims=(), collapsed_slice_dims=(0,),
                                    start_index_map=(0,))
    base0 = s * rpt

    def fetch_idx(i, buf):
        gix, dix, wv, s_g, s_d, s_w = buf
        row = base0 + i * RPC

        @pl.when(c == 0)
        def _i0():
            pltpu.async_copy(gidx0_hbm.at[pl.ds(row, RPC)], gix, s_g)

        @pl.when(c == 1)
        def _i1():
            pltpu.async_copy(gidx1_hbm.at[pl.ds(row, RPC)], gix, s_g)

        pltpu.async_copy(dst_hbm.at[pl.ds(row, RPC)], dix, s_d)
        base = pl.multiple_of(row * SUB, K)
        pltpu.async_copy(w_hbm.at[pl.ds(base, K)], wv, s_w)

    def wait_gix(buf):
        gix, _, _, s_g, _, _ = buf
        pltpu.make_async_copy(gidx0_hbm.at[pl.ds(0, RPC)], gix, s_g).wait()

    def wait_dw(buf):
        _, dix, wv, _, s_d, s_w = buf
        pltpu.make_async_copy(dst_hbm.at[pl.ds(0, RPC)], dix, s_d).wait()
        pltpu.make_async_copy(w_hbm.at[pl.ds(0, K)], wv, s_w).wait()

    def gather_slot(buf, q):
        gix = buf[0]
        pltpu.async_copy(y_hbm.at[gix.at[q]],
                         rows_v.at[pl.ds(q * SUB, SUB)], sg[q])

    def wait_slot(buf, q):
        gix = buf[0]
        pltpu.make_async_copy(y_hbm.at[gix.at[q]],
                              rows_v.at[pl.ds(q * SUB, SUB)], sg[q]).wait()

    def drain(cur, nxt):
        w_c = cur[2]
        dix_c = cur[1]
        for q in range(RPC):
            wait_slot(cur, q)

            def wgrp(j, carry2):
                wv = w_c[pl.ds(q * SUB + j * 16, 16)]
                for t in range(16):
                    e = q * SUB + j * 16 + t
                    wb = lax.gather(wv, jnp.full((16, 1), t, i32), dn, (1,),
                                    mode=lax.GatherScatterMode.PROMISE_IN_BOUNDS)
                    r0 = rows_v[e, pl.ds(0, 16)]
                    rows_v[e, pl.ds(0, 16)] = r0 * wb
                    r1 = rows_v[e, pl.ds(16, 16)]
                    rows_v[e, pl.ds(16, 16)] = r1 * wb
                return carry2

            lax.fori_loop(0, SUB // 16, wgrp, 0)
            pltpu.sync_copy(rows_v.at[pl.ds(q * SUB, SUB)],
                            agg_sh.at[dix_c.at[q]], add=True)
            if q == 0:
                wait_gix(nxt)
            gather_slot(nxt, q)

    fetch_idx(0, bufA)
    wait_gix(bufA)
    for q in range(RPC):
        gather_slot(bufA, q)
    fetch_idx(1, bufB)

    def two_chunks(j, carry):
        i = j * 2
        wait_dw(bufA)
        drain(bufA, bufB)
        fetch_idx(i + 2, bufA)
        wait_dw(bufB)
        drain(bufB, bufA)
        fetch_idx(i + 3, bufB)
        return carry

    lax.fori_loop(0, NCHUNK // 2, two_chunks, 0)

    wait_dw(bufA)
    wait_gix(bufB)
    wait_dw(bufB)
    for q in range(RPC):
        wait_slot(bufA, q)
    plsc.subcore_barrier()

    @pl.when(s < ZT)
    def _wb():
        pltpu.sync_copy(agg_sh.at[pl.ds(s * ZROWS, ZROWS)],
                        agg_out.at[c, pl.ds(s * ZROWS, ZROWS)])


_sc_agg = pl.kernel(
    _sc_agg_body,
    out_type=jax.ShapeDtypeStruct((NC, N, 32), f32),
    mesh=_mesh,
    scratch_types=[
        pltpu.VMEM_SHARED((NROWS, 32), f32),
        pltpu.VMEM((RPC, SUB), i32),
        pltpu.VMEM((RPC, SUB), i32),
        pltpu.VMEM((K,), f32),
        pltpu.VMEM((RPC, SUB), i32),
        pltpu.VMEM((RPC, SUB), i32),
        pltpu.VMEM((K,), f32),
        pltpu.VMEM((K, 32), f32),
        pltpu.VMEM((125, 32), f32),
        pltpu.SemaphoreType.DMA,
        pltpu.SemaphoreType.DMA,
        pltpu.SemaphoreType.DMA,
        pltpu.SemaphoreType.DMA,
        pltpu.SemaphoreType.DMA,
        pltpu.SemaphoreType.DMA,
        pltpu.SemaphoreType.DMA,
        pltpu.SemaphoreType.DMA,
        pltpu.SemaphoreType.DMA,
        pltpu.SemaphoreType.DMA,
    ],
    compiler_params=pltpu.CompilerParams(use_tc_tiling_on_sc=False),
)


def _tc_inv_body(c0_ref, c1_ref, inv_ref):
    inv_ref[...] = 1.0 / jnp.maximum(c0_ref[...] + c1_ref[...], 1.0)


_tc_inv = pl.pallas_call(
    _tc_inv_body,
    grid=(1,),
    in_specs=[
        pl.BlockSpec((TBL // 128, 128), lambda i: (0, 0)),
        pl.BlockSpec((TBL // 128, 128), lambda i: (0, 0)),
    ],
    out_specs=pl.BlockSpec((TBL // 128, 128), lambda i: (0, 0)),
    out_shape=jax.ShapeDtypeStruct((TBL // 128, 128), f32),
)


def _tc_layer1_body(nt_ref, emb_ref, a0_ref, a1_ref, root_ref, bias_ref,
                    y_ref, out0_ref):
    nt = nt_ref[...]
    oh = (nt == lax.broadcasted_iota(i32, (1, T), 1)).astype(f32)
    x = jnp.dot(oh, emb_ref[...], preferred_element_type=f32)
    y_ref[0] = jnp.dot(x, a0_ref[...], preferred_element_type=f32)
    y_ref[1] = jnp.dot(x, a1_ref[...], preferred_element_type=f32)
    out0_ref[...] = jnp.dot(x, root_ref[...], preferred_element_type=f32) + bias_ref[...]


_tc_layer1 = pl.pallas_call(
    _tc_layer1_body,
    grid=(NB,),
    in_specs=[
        pl.BlockSpec((BN, 1), lambda i: (i, 0)),
        pl.BlockSpec((T, H), lambda i: (0, 0)),
        pl.BlockSpec((H, R * 32), lambda i: (0, 0)),
        pl.BlockSpec((H, R * 32), lambda i: (0, 0)),
        pl.BlockSpec((H, H), lambda i: (0, 0)),
        pl.BlockSpec((1, H), lambda i: (0, 0)),
    ],
    out_specs=[
        pl.BlockSpec((2, BN, R * 32), lambda i: (0, i, 0)),
        pl.BlockSpec((BN, H), lambda i: (i, 0)),
    ],
    out_shape=[
        jax.ShapeDtypeStruct((2, N, R * 32), f32),
        jax.ShapeDtypeStruct((N, H), f32),
    ],
)


def _tc_layer2_body(prev_ref, g0_ref, g1_ref, a0_ref, a1_ref, root_ref,
                    bias_ref, y_ref, out0_ref):
    x = jax.nn.relu(prev_ref[...] +
                    jnp.concatenate([g0_ref[...], g1_ref[...]], axis=1))
    y_ref[0] = jnp.dot(x, a0_ref[...], preferred_element_type=f32)
    y_ref[1] = jnp.dot(x, a1_ref[...], preferred_element_type=f32)
    out0_ref[...] = jnp.dot(x, root_ref[...], preferred_element_type=f32) + bias_ref[...]


_tc_layer2 = pl.pallas_call(
    _tc_layer2_body,
    grid=(NB,),
    in_specs=[
        pl.BlockSpec((BN, H), lambda i: (i, 0)),
        pl.BlockSpec((BN, 32), lambda i: (i, 0)),
        pl.BlockSpec((BN, 32), lambda i: (i, 0)),
        pl.BlockSpec((H, R * 32), lambda i: (0, 0)),
        pl.BlockSpec((H, R * 32), lambda i: (0, 0)),
        pl.BlockSpec((H, H), lambda i: (0, 0)),
        pl.BlockSpec((1, H), lambda i: (0, 0)),
    ],
    out_specs=[
        pl.BlockSpec((2, BN, R * 32), lambda i: (0, i, 0)),
        pl.BlockSpec((BN, H), lambda i: (i, 0)),
    ],
    out_shape=[
        jax.ShapeDtypeStruct((2, N, R * 32), f32),
        jax.ShapeDtypeStruct((N, H), f32),
    ],
)


def _tc_head_body(prev_ref, g0_ref, g1_ref, b_ref, w1_ref, b1_ref, w2_ref,
                  b2_ref, logits_ref, psum, pcnt):
    i = pl.program_id(0)

    @pl.when(i == 0)
    def _init():
        psum[...] = jnp.zeros((G, H), f32)
        pcnt[...] = jnp.zeros((G, H), f32)

    x2 = jax.nn.relu(prev_ref[...] +
                     jnp.concatenate([g0_ref[...], g1_ref[...]], axis=1))
    oh = (b_ref[...] == lax.broadcasted_iota(i32, (1, G), 1)).astype(f32)
    psum[...] += lax.dot_general(oh, x2, (((0,), (0,)), ((), ())),
                                 preferred_element_type=f32)
    pcnt[...] += lax.dot_general(oh, jnp.ones((BN, H), f32),
                                 (((0,), (0,)), ((), ())),
                                 preferred_element_type=f32)

    @pl.when(i == NB - 1)
    def _fin():
        g = psum[...] / jnp.maximum(pcnt[...], 1.0)
        h = jax.nn.relu(jnp.dot(g, w1_ref[...], preferred_element_type=f32)
                        + b1_ref[...])
        logits_ref[...] = jnp.dot(h, w2_ref[...], preferred_element_type=f32) + b2_ref[...]


_tc_head = pl.pallas_call(
    _tc_head_body,
    grid=(NB,),
    in_specs=[
        pl.BlockSpec((BN, H), lambda i: (i, 0)),
        pl.BlockSpec((BN, 32), lambda i: (i, 0)),
        pl.BlockSpec((BN, 32), lambda i: (i, 0)),
        pl.BlockSpec((BN, 1), lambda i: (i, 0)),
        pl.BlockSpec((H, H), lambda i: (0, 0)),
        pl.BlockSpec((1, H), lambda i: (0, 0)),
        pl.BlockSpec((H, 128), lambda i: (0, 0)),
        pl.BlockSpec((1, 128), lambda i: (0, 0)),
    ],
    out_specs=pl.BlockSpec((G, 128), lambda i: (0, 0)),
    out_shape=jax.ShapeDtypeStruct((G, 128), f32),
    scratch_shapes=[
        pltpu.VMEM((G, H), f32),
        pltpu.VMEM((G, H), f32),
    ],
)


def kernel(emb, W, root, bias, W1, b1, W2, b2, node_type, edge_index,
           edge_type, batch):
    src = edge_index[0].astype(i32)
    dst = edge_index[1].astype(i32)
    et = edge_type.astype(i32)
    didx = dst * R + et
    gidx = src * R + et
    didx_p = jnp.concatenate(
        [didx, jnp.full((PADE,), N * R, i32)]).reshape(NROW_HBM, SUB)
    gidx_p = jnp.concatenate(
        [gidx, jnp.zeros((EPX - E,), i32)]).reshape(NROW_X, SUB)
    gidx_p1 = gidx_p + N * R
    dst_p = jnp.concatenate(
        [dst, jnp.full((EPX - E,), N, i32)]).reshape(NROW_X, SUB)

    cnt0, cnt1 = _sc_counts(didx_p)
    inv = _tc_inv(cnt0.reshape(TBL // 128, 128),
                  cnt1.reshape(TBL // 128, 128)).reshape(TBL)
    w = _sc_weights(didx_p, inv)
    w = jnp.concatenate([w, jnp.zeros((EPX - EP,), f32)])

    nt2 = node_type.reshape(N, 1).astype(i32)
    bat2 = batch.reshape(N, 1).astype(i32)
    W2p = jnp.zeros((H, 128), f32).at[:, :C].set(W2)
    b2p = jnp.zeros((1, 128), f32).at[0, :C].set(b2)
    A = [[W[l, :, :, cc * 32:(cc + 1) * 32].transpose(1, 0, 2).reshape(H, R * 32)
          for cc in range(2)] for l in range(2)]

    y, out0 = _tc_layer1(nt2, emb, A[0][0], A[0][1], root[0],
                         bias[0].reshape(1, H))
    agg = _sc_agg(y.reshape(2 * N * R, 32), gidx_p, gidx_p1, dst_p, w)
    yb, out0b = _tc_layer2(out0, agg[0], agg[1], A[1][0], A[1][1],
                           root[1], bias[1].reshape(1, H))
    agg2 = _sc_agg(yb.reshape(2 * N * R, 32), gidx_p, gidx_p1, dst_p, w)
    logits_p = _tc_head(out0b, agg2[0], agg2[1], bat2, W1,
                        b1.reshape(1, H), W2p, b2p)
    return logits_p[:, :C]
